# baseline (speedup 1.0000x reference)
import dataclasses
import functools

import jax
import jax.numpy as jnp
from jax import lax
from jax.experimental import pallas as pl
from jax.experimental.pallas import tpu as pltpu
from jax.experimental.pallas import tpu_sc as plsc

N = 10000
D = 128
E = 320000

NC = 2
NS = 16
NW = NC * NS
CHUNK = 128
EPT = 10240
NCHUNK = EPT // CHUNK
E_PAD = NW * EPT
CH0 = 160
PIECES = (64, 64, 32)
TCH = E_PAD // CHUNK
N_PAD = 10240
ZR = N_PAD // NS
NP2 = 10112
ZR2 = NP2 // NS

BN = 1000
NB = N // BN

_mesh = plsc.VectorSubcoreMesh(core_axis_name="c", subcore_axis_name="s")

_cp_no_layout = pltpu.CompilerParams()
if "needs_layout_passes" in pltpu.CompilerParams.__dataclass_fields__:
    _cp_no_layout = dataclasses.replace(_cp_no_layout, needs_layout_passes=False)


@functools.partial(
    pl.kernel,
    out_type=jax.ShapeDtypeStruct((NW, N_PAD), jnp.float32),
    mesh=_mesh,
    scratch_types=[
        pltpu.VMEM((EPT,), jnp.int32),
        pltpu.VMEM((N_PAD,), jnp.float32),
    ],
    compiler_params=_cp_no_layout,
)
def _sc_degree(dst_hbm, deg_hbm, dstv, hist):
    cid = lax.axis_index("c")
    sid = lax.axis_index("s")
    wid = sid * NC + cid
    pltpu.sync_copy(dst_hbm.at[wid], dstv)
    zeros16 = jnp.zeros((16,), jnp.float32)
    ones16 = jnp.ones((16,), jnp.float32)

    @pl.loop(0, N_PAD // 16)
    def _(i):
        hist[pl.ds(i * 16, 16)] = zeros16

    @pl.loop(0, EPT // 16)
    def _(i):
        idx = dstv[pl.ds(i * 16, 16)]
        plsc.addupdate_scatter(hist, [idx], ones16)

    pltpu.sync_copy(hist, deg_hbm.at[wid])


@functools.partial(
    pl.kernel,
    out_type=jax.ShapeDtypeStruct((NP2, D), jnp.float32),
    mesh=_mesh,
    scratch_types=[
        pltpu.VMEM((64, CHUNK), jnp.int32),
        pltpu.VMEM((64, CHUNK), jnp.int32),
        pltpu.VMEM((CHUNK, D), jnp.float32),
        pltpu.VMEM((CHUNK, D), jnp.float32),
        pltpu.VMEM_SHARED((NP2, D), jnp.float32),
        pltpu.SemaphoreType.DMA,
        pltpu.SemaphoreType.DMA,
    ],
)
def _sc_scatter(g_hbm, src_hbm, dst_hbm, zeros_hbm, acc_hbm,
                srcv, dstv, r0, r1, acc, sg, ss):
    cid = lax.axis_index("c")
    sid = lax.axis_index("s")
    gref = g_hbm
    zref = zeros_hbm

    @pl.when(cid == 0)
    def _():
        pltpu.sync_copy(zref, acc.at[pl.ds(sid * ZR2, ZR2)])

    plsc.subcore_barrier()

    def _gather(c, rows):
        pltpu.async_copy(gref.at[srcv.at[c]], rows, sg)

    def _gwait(c, rows):
        pltpu.make_async_copy(gref.at[srcv.at[c]], rows, sg).wait()

    def _scat(c, rows):
        pltpu.async_copy(rows, acc.at[dstv.at[c]], ss, add=True)

    def _swait(rows):
        pltpu.make_async_copy(zref.at[pl.ds(0, CHUNK)], rows, ss).wait()

    def _run(base):
        off = 0
        for half in PIECES:
            pltpu.sync_copy(src_hbm.at[pl.ds(base + off, half)],
                            srcv.at[pl.ds(0, half)])
            pltpu.sync_copy(dst_hbm.at[pl.ds(base + off, half)],
                            dstv.at[pl.ds(0, half)])
            off += half
            _gather(0, r0)

            @pl.loop(0, half, step=2)
            def _(c):
                _gwait(c, r0)
                _scat(c, r0)
                _gather(c + 1, r1)
                _swait(r0)
                _gwait(c + 1, r1)
                _scat(c + 1, r1)

                @pl.when(c + 2 < half)
                def _():
                    _gather(c + 2, r0)

                _swait(r1)

    @pl.when(cid == 0)
    def _():
        _run(sid * CH0)

    plsc.subcore_barrier()

    @pl.when(cid == 0)
    def _():
        pltpu.sync_copy(acc.at[pl.ds(sid * ZR2, ZR2)],
                        acc_hbm.at[pl.ds(sid * ZR2, ZR2)])


def _disk_body(deg_ref, dis_ref):
    dsum = jnp.sum(deg_ref[...], axis=0)[:, None] + 1.0
    dis_ref[...] = lax.rsqrt(dsum)


def _mat0_body(x_ref, w_ref, h_ref):
    h_ref[...] = jnp.dot(x_ref[...], w_ref[...],
                         preferred_element_type=jnp.float32)


def _scale_body(dis_ref, h_ref, g_ref):
    g_ref[...] = h_ref[...] * dis_ref[...]


def _combine_body(dis_ref, acc_ref, g_ref, b_ref, w_ref, gout_ref):
    dis = dis_ref[...]
    s = acc_ref[...] + g_ref[...]
    xl = jnp.maximum(s * dis + b_ref[...], 0.0)
    h = jnp.dot(xl, w_ref[...], preferred_element_type=jnp.float32)
    gout_ref[...] = h * dis


def _final_body(dis_ref, acc_ref, g_ref, b_ref, out_ref):
    dis = dis_ref[...]
    s = acc_ref[...] + g_ref[...]
    out_ref[...] = jnp.maximum(s * dis + b_ref[...], 0.0)


BD = 2048


def _disk(deg):
    return pl.pallas_call(
        _disk_body, grid=(N_PAD // BD,),
        in_specs=[pl.BlockSpec((NW, BD), lambda i: (0, i))],
        out_specs=pl.BlockSpec((BD, 1), lambda i: (i, 0)),
        out_shape=jax.ShapeDtypeStruct((N_PAD, 1), jnp.float32),
    )(deg)


_dis_spec = pl.BlockSpec((BN, 1), lambda i: (i, 0))
_g2_spec = pl.BlockSpec((NC, BN, D), lambda i: (0, i, 0))
_g2_f32 = jax.ShapeDtypeStruct((NC, N, D), jnp.float32)
_acc_spec = pl.BlockSpec((BN, D), lambda i: (i, 0))
_row_spec = pl.BlockSpec((BN, D), lambda i: (i, 0))
_w_spec = pl.BlockSpec((D, D), lambda i: (0, 0))
_b_spec = pl.BlockSpec((1, D), lambda i: (0, 0))
_nd_f32 = jax.ShapeDtypeStruct((N, D), jnp.float32)


def _mat0(x, W):
    return pl.pallas_call(
        _mat0_body, grid=(NB,),
        in_specs=[_row_spec, _w_spec],
        out_specs=_row_spec, out_shape=_nd_f32,
    )(x, W)


def _scale(dis, h):
    return pl.pallas_call(
        _scale_body, grid=(NB,),
        in_specs=[_dis_spec, _row_spec],
        out_specs=_row_spec, out_shape=_nd_f32,
    )(dis, h)


def _combine(dis, accs, g, b, W):
    return pl.pallas_call(
        _combine_body, grid=(NB,),
        in_specs=[_dis_spec, _acc_spec, _row_spec, _b_spec, _w_spec],
        out_specs=_row_spec, out_shape=_nd_f32,
    )(dis, accs, g, b, W)


def _final(dis, accs, g, b):
    return pl.pallas_call(
        _final_body, grid=(NB,),
        in_specs=[_dis_spec, _acc_spec, _row_spec, _b_spec],
        out_specs=_row_spec, out_shape=_nd_f32,
    )(dis, accs, g, b)


def kernel(x, edge_index, W0, W1, W2, b0, b1, b2):
    src = edge_index[0]
    dst = edge_index[1]
    pad = E_PAD - E
    srcR = jnp.concatenate(
        [src, jnp.zeros((pad,), src.dtype)]).reshape(TCH, CHUNK)
    dstR = jnp.concatenate(
        [dst, jnp.full((pad,), N, dst.dtype)]).reshape(TCH, CHUNK)
    zeros_d = jnp.zeros((ZR2, D), jnp.float32)
    dstF = dstR.reshape(NW, EPT)

    deg = _sc_degree(dstF)
    h0 = _mat0(x, W0)
    dis = _disk(deg)
    g = _scale(dis, h0)
    for W, b in ((W1, b0), (W2, b1)):
        accs = _sc_scatter(g, srcR, dstR, zeros_d)
        g = _combine(dis, accs, g, b.reshape(1, D), W)
    accs = _sc_scatter(g, srcR, dstR, zeros_d)
    return _final(dis, accs, g, b2.reshape(1, D))


# speedup vs baseline: 1.4089x; 1.4089x over previous
import dataclasses
import functools

import jax
import jax.numpy as jnp
from jax import lax
from jax.experimental import pallas as pl
from jax.experimental.pallas import tpu as pltpu
from jax.experimental.pallas import tpu_sc as plsc

N = 10000
D = 128
E = 320000

NC = 2
NS = 16
NW = NC * NS
CHUNK = 128
EPT = 10240
NCHUNK = EPT // CHUNK
E_PAD = NW * EPT
CH0 = 128
CH1 = 32
TCH = E_PAD // CHUNK
N_PAD = 10240
ZR = N_PAD // NS

BN = 1000
NB = N // BN

_mesh = plsc.VectorSubcoreMesh(core_axis_name="c", subcore_axis_name="s")

_cp_no_layout = pltpu.CompilerParams()
if "needs_layout_passes" in pltpu.CompilerParams.__dataclass_fields__:
    _cp_no_layout = dataclasses.replace(_cp_no_layout, needs_layout_passes=False)


@functools.partial(
    pl.kernel,
    out_type=jax.ShapeDtypeStruct((NW, N_PAD), jnp.float32),
    mesh=_mesh,
    scratch_types=[
        pltpu.VMEM((EPT,), jnp.int32),
        pltpu.VMEM((N_PAD,), jnp.float32),
    ],
    compiler_params=_cp_no_layout,
)
def _sc_degree(dst_hbm, deg_hbm, dstv, hist):
    cid = lax.axis_index("c")
    sid = lax.axis_index("s")
    wid = sid * NC + cid
    pltpu.sync_copy(dst_hbm.at[wid], dstv)
    zeros16 = jnp.zeros((16,), jnp.float32)
    ones16 = jnp.ones((16,), jnp.float32)

    @pl.loop(0, N_PAD // 16)
    def _(i):
        hist[pl.ds(i * 16, 16)] = zeros16

    @pl.loop(0, EPT // 16)
    def _(i):
        idx = dstv[pl.ds(i * 16, 16)]
        plsc.addupdate_scatter(hist, [idx], ones16)

    pltpu.sync_copy(hist, deg_hbm.at[wid])


@functools.partial(
    pl.kernel,
    out_type=jax.ShapeDtypeStruct((NC, N_PAD, D), jnp.float32),
    mesh=_mesh,
    scratch_types=[
        pltpu.VMEM((CH0 // 2, CHUNK), jnp.int32),
        pltpu.VMEM((CH0 // 2, CHUNK), jnp.int32),
        pltpu.VMEM((CHUNK, D), jnp.float32),
        pltpu.VMEM((CHUNK, D), jnp.float32),
        pltpu.VMEM_SHARED((N_PAD, D), jnp.float32),
        pltpu.SemaphoreType.DMA,
        pltpu.SemaphoreType.DMA,
    ],
)
def _sc_scatter(g_hbm, src_hbm, dst_hbm, zeros_hbm, acc_hbm,
                srcv, dstv, r0, r1, acc, sg, ss):
    cid = lax.axis_index("c")
    sid = lax.axis_index("s")
    gref = g_hbm.at[cid]
    zref = zeros_hbm.at[cid]
    pltpu.sync_copy(zref, acc.at[pl.ds(sid * ZR, ZR)])
    plsc.subcore_barrier()

    def _gather(c, rows):
        pltpu.async_copy(gref.at[srcv.at[c]], rows, sg)

    def _gwait(c, rows):
        pltpu.make_async_copy(gref.at[srcv.at[c]], rows, sg).wait()

    def _scat(c, rows):
        pltpu.async_copy(rows, acc.at[dstv.at[c]], ss, add=True)

    def _swait(rows):
        pltpu.make_async_copy(zref.at[pl.ds(0, CHUNK)], rows, ss).wait()

    def _run(base, nch):
        half = nch // 2
        for h in range(2):
            pltpu.sync_copy(src_hbm.at[pl.ds(base + h * half, half)],
                            srcv.at[pl.ds(0, half)])
            pltpu.sync_copy(dst_hbm.at[pl.ds(base + h * half, half)],
                            dstv.at[pl.ds(0, half)])
            _gather(0, r0)

            @pl.loop(0, half, step=2)
            def _(c):
                _gwait(c, r0)
                _scat(c, r0)
                _gather(c + 1, r1)
                _swait(r0)
                _gwait(c + 1, r1)
                _scat(c + 1, r1)

                @pl.when(c + 2 < half)
                def _():
                    _gather(c + 2, r0)

                _swait(r1)

    @pl.when(cid == 0)
    def _():
        _run(sid * CH0, CH0)

    @pl.when(cid == 1)
    def _():
        _run(NS * CH0 + sid * CH1, CH1)

    plsc.subcore_barrier()
    pltpu.sync_copy(acc.at[pl.ds(sid * ZR, ZR)],
                    acc_hbm.at[cid, pl.ds(sid * ZR, ZR)])


def _disk_body(deg_ref, dis_ref):
    dsum = jnp.sum(deg_ref[...], axis=0)[:, None] + 1.0
    dis_ref[...] = lax.rsqrt(dsum)


def _mat0_body(x_ref, w_ref, h_ref):
    h_ref[...] = jnp.dot(x_ref[...], w_ref[...],
                         preferred_element_type=jnp.float32)


def _scale_body(dis_ref, h_ref, g_ref):
    v = h_ref[...] * dis_ref[...]
    g_ref[0] = v
    g_ref[1] = v


def _combine_body(dis_ref, acc_ref, g_ref, b_ref, w_ref, gout_ref):
    dis = dis_ref[...]
    s = acc_ref[0] + acc_ref[1] + g_ref[0]
    xl = jnp.maximum(s * dis + b_ref[...], 0.0)
    h = jnp.dot(xl, w_ref[...], preferred_element_type=jnp.float32)
    v = h * dis
    gout_ref[0] = v
    gout_ref[1] = v


def _final_body(dis_ref, acc_ref, g_ref, b_ref, out_ref):
    dis = dis_ref[...]
    s = acc_ref[0] + acc_ref[1] + g_ref[0]
    out_ref[...] = jnp.maximum(s * dis + b_ref[...], 0.0)


BD = 2048


def _disk(deg):
    return pl.pallas_call(
        _disk_body, grid=(N_PAD // BD,),
        in_specs=[pl.BlockSpec((NW, BD), lambda i: (0, i))],
        out_specs=pl.BlockSpec((BD, 1), lambda i: (i, 0)),
        out_shape=jax.ShapeDtypeStruct((N_PAD, 1), jnp.float32),
    )(deg)


_dis_spec = pl.BlockSpec((BN, 1), lambda i: (i, 0))
_g2_spec = pl.BlockSpec((NC, BN, D), lambda i: (0, i, 0))
_g2_f32 = jax.ShapeDtypeStruct((NC, N, D), jnp.float32)
_acc_spec = pl.BlockSpec((NC, BN, D), lambda i: (0, i, 0))
_row_spec = pl.BlockSpec((BN, D), lambda i: (i, 0))
_w_spec = pl.BlockSpec((D, D), lambda i: (0, 0))
_b_spec = pl.BlockSpec((1, D), lambda i: (0, 0))
_nd_f32 = jax.ShapeDtypeStruct((N, D), jnp.float32)


def _mat0(x, W):
    return pl.pallas_call(
        _mat0_body, grid=(NB,),
        in_specs=[_row_spec, _w_spec],
        out_specs=_row_spec, out_shape=_nd_f32,
    )(x, W)


def _scale(dis, h):
    return pl.pallas_call(
        _scale_body, grid=(NB,),
        in_specs=[_dis_spec, _row_spec],
        out_specs=_g2_spec, out_shape=_g2_f32,
    )(dis, h)


def _combine(dis, accs, g, b, W):
    return pl.pallas_call(
        _combine_body, grid=(NB,),
        in_specs=[_dis_spec, _acc_spec, _g2_spec, _b_spec, _w_spec],
        out_specs=_g2_spec, out_shape=_g2_f32,
    )(dis, accs, g, b, W)


def _final(dis, accs, g, b):
    return pl.pallas_call(
        _final_body, grid=(NB,),
        in_specs=[_dis_spec, _acc_spec, _g2_spec, _b_spec],
        out_specs=_row_spec, out_shape=_nd_f32,
    )(dis, accs, g, b)


def kernel(x, edge_index, W0, W1, W2, b0, b1, b2):
    src = edge_index[0]
    dst = edge_index[1]
    pad = E_PAD - E
    srcR = jnp.concatenate(
        [src, jnp.zeros((pad,), src.dtype)]).reshape(TCH, CHUNK)
    dstR = jnp.concatenate(
        [dst, jnp.full((pad,), N, dst.dtype)]).reshape(TCH, CHUNK)
    zeros_d = jnp.zeros((NC, ZR, D), jnp.float32)
    dstF = dstR.reshape(NW, EPT)

    deg = _sc_degree(dstF)
    h0 = _mat0(x, W0)
    dis = _disk(deg)
    g = _scale(dis, h0)
    for W, b in ((W1, b0), (W2, b1)):
        accs = _sc_scatter(g, srcR, dstR, zeros_d)
        g = _combine(dis, accs, g, b.reshape(1, D), W)
    accs = _sc_scatter(g, srcR, dstR, zeros_d)
    return _final(dis, accs, g, b2.reshape(1, D))
